# Initial kernel scaffold; baseline (speedup 1.0000x reference)
import jax
import jax.numpy as jnp
from jax.experimental import pallas as pl


def kernel(inputs, table):
    raise NotImplementedError("write your pallas kernel here")


# speedup vs baseline: 1.1075x; 1.1075x over previous
import functools

import jax
import jax.numpy as jnp
from jax import lax
from jax.experimental import pallas as pl
from jax.experimental.pallas import tpu as pltpu
from jax.experimental.pallas import tpu_sc as plsc

_NUM_CORES = 2
_NUM_SUBCORES = 16
_NW = _NUM_CORES * _NUM_SUBCORES


@functools.lru_cache(maxsize=None)
def _make_gather(n_rows: int, vocab: int, dim: int):
    assert n_rows % _NW == 0
    b_per_w = n_rows // _NW
    chunk = 1600
    assert b_per_w % chunk == 0
    n_chunks = b_per_w // chunk

    mesh = plsc.VectorSubcoreMesh(core_axis_name="c", subcore_axis_name="s")

    @functools.partial(
        pl.kernel,
        mesh=mesh,
        compiler_params=pltpu.CompilerParams(use_tc_tiling_on_sc=False),
        out_type=jax.ShapeDtypeStruct((n_rows, dim), jnp.float32),
        scratch_types=[
            pltpu.VMEM((b_per_w,), jnp.int32),
            pltpu.VMEM((chunk, dim), jnp.float32),
            pltpu.SemaphoreType.DMA,
        ],
    )
    def gather_kernel(table_hbm, idx_hbm, out_hbm, idx_v, rows_v, sem):
        wid = lax.axis_index("s") * _NUM_CORES + lax.axis_index("c")
        base = wid * b_per_w
        pltpu.sync_copy(idx_hbm.at[pl.ds(base, b_per_w)], idx_v)

        def body(c, carry):
            off = c * chunk
            pltpu.async_copy(
                table_hbm.at[idx_v.at[pl.ds(off, chunk)]], rows_v, sem
            ).wait()
            pltpu.sync_copy(rows_v, out_hbm.at[pl.ds(base + off, chunk)])
            return carry

        lax.fori_loop(0, n_chunks, body, 0)

    return gather_kernel


def kernel(inputs, table):
    b, l = inputs.shape
    vocab, dim = table.shape
    flat_idx = inputs.reshape(b * l)
    out = _make_gather(b * l, vocab, dim)(table, flat_idx)
    return out.reshape(b, l, dim)


# speedup vs baseline: 1.3814x; 1.2472x over previous
import functools

import jax
import jax.numpy as jnp
from jax import lax
from jax.experimental import pallas as pl
from jax.experimental.pallas import tpu as pltpu
from jax.experimental.pallas import tpu_sc as plsc

_NUM_CORES = 2
_NUM_SUBCORES = 16
_NW = _NUM_CORES * _NUM_SUBCORES
_LANES = 16


@functools.lru_cache(maxsize=None)
def _make_gather(batch: int, seq: int, vocab: int, dim: int):
    assert batch % _NW == 0
    b_per_w = batch // _NW
    n_blk = b_per_w // _LANES
    assert dim == 32

    mesh = plsc.VectorSubcoreMesh(core_axis_name="c", subcore_axis_name="s")

    @functools.partial(
        pl.kernel,
        mesh=mesh,
        compiler_params=pltpu.CompilerParams(
            use_tc_tiling_on_sc=False, needs_layout_passes=False
        ),
        out_type=jax.ShapeDtypeStruct((seq, dim, batch), jnp.float32),
        scratch_types=[
            pltpu.VMEM((b_per_w * seq,), jnp.int32),
            pltpu.VMEM((b_per_w,), jnp.int32),
            pltpu.VMEM((b_per_w, dim), jnp.float32),
            pltpu.VMEM((dim, b_per_w), jnp.float32),
            pltpu.SemaphoreType.DMA,
        ],
    )
    def gather_kernel(table_hbm, idx_hbm, out_hbm, idx_all, idx_l, rows_v, t_v, sem):
        wid = lax.axis_index("s") * _NUM_CORES + lax.axis_index("c")
        b0 = wid * b_per_w
        pltpu.sync_copy(idx_hbm.at[pl.ds(b0 * seq, b_per_w * seq)], idx_all)
        iota = lax.iota(jnp.int32, _LANES)

        def per_l(l, carry):
            def ext(jb, c):
                pos = (iota + jb * _LANES) * seq + l
                idx_l[pl.ds(jb * _LANES, _LANES)] = plsc.load_gather(idx_all, [pos])
                return c

            lax.fori_loop(0, n_blk, ext, 0, unroll=4)

            pltpu.async_copy(table_hbm.at[idx_l], rows_v, sem).wait()

            def tr(jb, c):
                rvec = iota + jb * _LANES
                for d in range(dim):
                    col = jnp.full((_LANES,), d, jnp.int32)
                    t_v[d, pl.ds(jb * _LANES, _LANES)] = plsc.load_gather(
                        rows_v, [rvec, col]
                    )
                return c

            lax.fori_loop(0, n_blk, tr, 0)

            pltpu.sync_copy(t_v, out_hbm.at[l, :, pl.ds(b0, b_per_w)])
            return carry

        lax.fori_loop(0, seq, per_l, 0)

    return gather_kernel


def kernel(inputs, table):
    b, l = inputs.shape
    vocab, dim = table.shape
    flat_idx = inputs.reshape(b * l)
    out = _make_gather(b, l, vocab, dim)(table, flat_idx)
    return out.transpose(2, 0, 1)


# speedup vs baseline: 1.4657x; 1.0610x over previous
import functools

import jax
import jax.numpy as jnp
from jax import lax
from jax.experimental import pallas as pl
from jax.experimental.pallas import tpu as pltpu
from jax.experimental.pallas import tpu_sc as plsc

_NUM_CORES = 2
_NUM_SUBCORES = 16
_NW = _NUM_CORES * _NUM_SUBCORES
_LANES = 16


@functools.lru_cache(maxsize=None)
def _make_gather(batch: int, seq: int, vocab: int, dim: int):
    assert batch % _NW == 0
    b_per_w = batch // _NW
    n_blk = b_per_w // _LANES
    assert dim == 32
    assert seq % 2 == 0

    mesh = plsc.VectorSubcoreMesh(core_axis_name="c", subcore_axis_name="s")

    @functools.partial(
        pl.kernel,
        mesh=mesh,
        compiler_params=pltpu.CompilerParams(
            use_tc_tiling_on_sc=False, needs_layout_passes=False
        ),
        out_type=jax.ShapeDtypeStruct((seq, dim, batch), jnp.float32),
        scratch_types=[
            pltpu.VMEM((b_per_w,), jnp.int32),
            pltpu.VMEM((b_per_w,), jnp.int32),
            pltpu.VMEM((b_per_w, dim), jnp.float32),
            pltpu.VMEM((b_per_w, dim), jnp.float32),
            pltpu.VMEM((dim, b_per_w), jnp.float32),
            pltpu.VMEM((dim, b_per_w), jnp.float32),
            pltpu.SemaphoreType.DMA,
            pltpu.SemaphoreType.DMA,
            pltpu.SemaphoreType.DMA,
            pltpu.SemaphoreType.DMA,
        ],
    )
    def gather_kernel(
        table_hbm, idx_hbm, out_hbm,
        idx0, idx1, rows0, rows1, tv0, tv1, gsem0, gsem1, wsem0, wsem1,
    ):
        idx_bufs = (idx0, idx1)
        rows = (rows0, rows1)
        tvs = (tv0, tv1)
        gsems = (gsem0, gsem1)
        wsems = (wsem0, wsem1)

        wid = lax.axis_index("s") * _NUM_CORES + lax.axis_index("c")
        b0 = wid * b_per_w
        iota = lax.iota(jnp.int32, _LANES)

        def start(l, k):
            pltpu.sync_copy(idx_hbm.at[pl.ds(l * batch + b0, b_per_w)], idx_bufs[k])
            pltpu.async_copy(table_hbm.at[idx_bufs[k]], rows[k], gsems[k])

        start(0, 0)

        def body(i, carry):
            for k in range(2):
                l = 2 * i + k
                if k == 0:
                    start(l + 1, 1)
                else:
                    @pl.when(i < seq // 2 - 1)
                    def _():
                        start(l + 1, 0)

                pltpu.make_async_copy(
                    table_hbm.at[idx_bufs[k]], rows[k], gsems[k]
                ).wait()

                @pl.when(i >= 1)
                def _():
                    pltpu.make_async_copy(
                        tvs[k], out_hbm.at[l, :, pl.ds(b0, b_per_w)], wsems[k]
                    ).wait()

                def tr(jb, c):
                    rvec = iota + jb * _LANES
                    for d in range(dim):
                        col = jnp.full((_LANES,), d, jnp.int32)
                        tvs[k][d, pl.ds(jb * _LANES, _LANES)] = plsc.load_gather(
                            rows[k], [rvec, col]
                        )
                    return c

                lax.fori_loop(0, n_blk, tr, 0)

                pltpu.async_copy(
                    tvs[k], out_hbm.at[l, :, pl.ds(b0, b_per_w)], wsems[k]
                )
            return carry

        lax.fori_loop(0, seq // 2, body, 0)

        for k in range(2):
            pltpu.make_async_copy(
                tvs[k], out_hbm.at[0, :, pl.ds(b0, b_per_w)], wsems[k]
            ).wait()

    return gather_kernel


def kernel(inputs, table):
    b, l = inputs.shape
    vocab, dim = table.shape
    flat_idx = inputs.T.reshape(b * l)
    out = _make_gather(b, l, vocab, dim)(table, flat_idx)
    return out.transpose(2, 0, 1)


# speedup vs baseline: 2.2076x; 1.5062x over previous
import functools

import jax
import jax.numpy as jnp
from jax import lax
from jax.experimental import pallas as pl
from jax.experimental.pallas import tpu as pltpu
from jax.experimental.pallas import tpu_sc as plsc

_NUM_CORES = 2
_NUM_SUBCORES = 16
_NW = _NUM_CORES * _NUM_SUBCORES
_LANES = 16


@functools.lru_cache(maxsize=None)
def _make_gather(batch: int, seq: int, vocab: int, dim: int):
    assert batch % _NW == 0
    b_per_w = batch // _NW
    n_blk = b_per_w // _LANES
    assert dim == 32
    assert seq % 2 == 0

    mesh = plsc.VectorSubcoreMesh(core_axis_name="c", subcore_axis_name="s")

    @functools.partial(
        pl.kernel,
        mesh=mesh,
        compiler_params=pltpu.CompilerParams(
            use_tc_tiling_on_sc=False, needs_layout_passes=False
        ),
        out_type=jax.ShapeDtypeStruct((seq, dim, batch), jnp.float32),
        scratch_types=[
            pltpu.VMEM((b_per_w,), jnp.int32),
            pltpu.VMEM((b_per_w,), jnp.int32),
            pltpu.VMEM((b_per_w, dim), jnp.float32),
            pltpu.VMEM((b_per_w, dim), jnp.float32),
            pltpu.VMEM((dim, b_per_w + 1), jnp.float32),
            pltpu.VMEM((dim, b_per_w + 1), jnp.float32),
            pltpu.SemaphoreType.DMA,
            pltpu.SemaphoreType.DMA,
            pltpu.SemaphoreType.DMA,
            pltpu.SemaphoreType.DMA,
        ],
    )
    def gather_kernel(
        table_hbm, idx_hbm, out_hbm,
        idx0, idx1, rows0, rows1, tv0, tv1, gsem0, gsem1, wsem0, wsem1,
    ):
        idx_bufs = (idx0, idx1)
        rows = (rows0, rows1)
        tvs = (tv0, tv1)
        gsems = (gsem0, gsem1)
        wsems = (wsem0, wsem1)

        wid = lax.axis_index("s") * _NUM_CORES + lax.axis_index("c")
        b0 = wid * b_per_w
        iota = lax.iota(jnp.int32, _LANES)

        def start(l, k):
            pltpu.sync_copy(idx_hbm.at[pl.ds(l * batch + b0, b_per_w)], idx_bufs[k])
            pltpu.async_copy(table_hbm.at[idx_bufs[k]], rows[k], gsems[k])

        start(0, 0)

        def body(i, carry):
            for k in range(2):
                l = 2 * i + k
                if k == 0:
                    start(l + 1, 1)
                else:
                    @pl.when(i < seq // 2 - 1)
                    def _():
                        start(l + 1, 0)

                pltpu.make_async_copy(
                    table_hbm.at[idx_bufs[k]], rows[k], gsems[k]
                ).wait()

                @pl.when(i >= 1)
                def _():
                    pltpu.make_async_copy(
                        tvs[k].at[:, pl.ds(0, b_per_w)],
                        out_hbm.at[l, :, pl.ds(b0, b_per_w)],
                        wsems[k],
                    ).wait()

                def tr(jb, c):
                    for rr in range(8):
                        r = jb * 8 + rr
                        rsplat = jnp.full((_LANES,), r, jnp.int32)
                        for d0 in range(0, dim, _LANES):
                            v = rows[k][r, pl.ds(d0, _LANES)]
                            plsc.store_scatter(tvs[k], [iota + d0, rsplat], v)
                    return c

                lax.fori_loop(0, b_per_w // 8, tr, 0)

                pltpu.async_copy(
                    tvs[k].at[:, pl.ds(0, b_per_w)],
                    out_hbm.at[l, :, pl.ds(b0, b_per_w)],
                    wsems[k],
                )
            return carry

        lax.fori_loop(0, seq // 2, body, 0)

        for k in range(2):
            pltpu.make_async_copy(
                tvs[k].at[:, pl.ds(0, b_per_w)],
                out_hbm.at[0, :, pl.ds(b0, b_per_w)],
                wsems[k],
            ).wait()

    return gather_kernel


def kernel(inputs, table):
    b, l = inputs.shape
    vocab, dim = table.shape
    flat_idx = inputs.T.reshape(b * l)
    out = _make_gather(b, l, vocab, dim)(table, flat_idx)
    return out.transpose(2, 0, 1)
